# baseline (speedup 1.0000x reference)
import jax
import jax.numpy as jnp
from jax.experimental import pallas as pl
from jax.experimental.pallas import tpu as pltpu
from jax.experimental.pallas import tpu_sc as plsc

B = 16384
VOCAB = 1000
EMB = 8
NWIDE = 8
NDEEP = 4
NDENSE = 4
NSLOT = NWIDE + NDEEP
NCLS = 2
PCOLS = NSLOT * NCLS

NC = 2
NS = 16
L = 16
NW = NC * NS
BPW = B // NW
NG = BPW // L


def _r(x):
    return x.astype(jnp.bfloat16).astype(jnp.float32)


def _project_body(emb_ref, w_ref, b_ref, xs_ref, wd_ref, p_ref, d_ref):
    e = _r(emb_ref[...])
    w = _r(w_ref[...])
    acc = b_ref[...] + e[:, 0:1] * w[0:1, :]
    for k in range(1, EMB):
        acc = acc + e[:, k:k + 1] * w[k:k + 1, :]
    p_ref[...] = acc
    xs = _r(xs_ref[...])
    wd = _r(wd_ref[...])
    d = xs[:, 0:1] * wd[0:1, :]
    for k in range(1, NDENSE):
        d = d + xs[:, k:k + 1] * wd[k:k + 1, :]
    d_ref[...] = d


def _sc_body(pf_hbm, xw_hbm, xd_hbm, dv_hbm,
             lg_hbm, tg_hbm, pb_hbm,
             pf, xw, xd, dv, lg, tg, pb):
    wid = jax.lax.axis_index("s") * NC + jax.lax.axis_index("c")
    base = wid * BPW

    pltpu.sync_copy(pf_hbm, pf)
    pltpu.sync_copy(xw_hbm.at[pl.ds(base, BPW)], xw)
    pltpu.sync_copy(xd_hbm.at[pl.ds(base, BPW)], xd)
    pltpu.sync_copy(dv_hbm.at[pl.ds(base, BPW)], dv)

    iw = jnp.arange(L, dtype=jnp.int32)
    zer = jnp.zeros((L,), jnp.int32)
    one = zer + 1

    def group(g, carry):
        row = g * L + iw
        acc0 = plsc.load_gather(dv, [row, zer])
        acc1 = plsc.load_gather(dv, [row, one])
        for s in range(NWIDE):
            idx = plsc.load_gather(xw, [row, zer + s])
            acc0 = acc0 + plsc.load_gather(pf, [idx, zer + (2 * s)])
            acc1 = acc1 + plsc.load_gather(pf, [idx, zer + (2 * s + 1)])
        for s in range(NDEEP):
            idx = plsc.load_gather(xd, [row, zer + s])
            acc0 = acc0 + plsc.load_gather(pf, [idx, zer + (2 * (NWIDE + s))])
            acc1 = acc1 + plsc.load_gather(pf, [idx, zer + (2 * (NWIDE + s) + 1)])
        plsc.store_scatter(lg, [row, zer], acc0)
        plsc.store_scatter(lg, [row, one], acc1)
        m = jnp.maximum(acc0, acc1)
        e0 = jnp.exp(acc0 - m)
        e1 = jnp.exp(acc1 - m)
        inv = 1.0 / (e0 + e1)
        plsc.store_scatter(pb, [row, zer], e0 * inv)
        plsc.store_scatter(pb, [row, one], e1 * inv)
        t = jnp.where(acc1 > acc0, 1, 0).astype(jnp.int32)
        plsc.store_scatter(tg, [row, zer], t)
        return carry

    jax.lax.fori_loop(0, NG, group, 0)

    pltpu.sync_copy(lg, lg_hbm.at[pl.ds(base, BPW)])
    pltpu.sync_copy(tg, tg_hbm.at[pl.ds(base, BPW)])
    pltpu.sync_copy(pb, pb_hbm.at[pl.ds(base, BPW)])


def kernel(x_wide, x_deep, x_dense, emb, fc_w, fc_b):
    x_wide = x_wide.astype(jnp.int32)
    x_deep = x_deep.astype(jnp.int32)
    x_dense = x_dense.astype(jnp.float32)
    emb = emb.astype(jnp.float32)
    fc_w = fc_w.astype(jnp.float32)
    fc_b = fc_b.astype(jnp.float32)

    w_proj = (
        fc_w[:, : NSLOT * EMB]
        .reshape(NCLS, NSLOT, EMB)
        .transpose(2, 1, 0)
        .reshape(EMB, PCOLS)
    )
    bias_row = jnp.concatenate(
        [fc_b, jnp.zeros((PCOLS - NCLS,), jnp.float32)]
    )[None, :]
    wd = fc_w[:, NSLOT * EMB:].T

    p_tab, dmat = pl.pallas_call(
        _project_body,
        out_shape=[
            jax.ShapeDtypeStruct((VOCAB, PCOLS), jnp.float32),
            jax.ShapeDtypeStruct((B, NCLS), jnp.float32),
        ],
    )(emb, w_proj, bias_row, x_dense, wd)

    mesh = plsc.VectorSubcoreMesh(
        core_axis_name="c", subcore_axis_name="s",
        num_cores=NC, num_subcores=NS,
    )
    sc = pl.kernel(
        _sc_body,
        compiler_params=pltpu.CompilerParams(
            needs_layout_passes=False, use_tc_tiling_on_sc=False),
        out_type=[
            jax.ShapeDtypeStruct((B, NCLS), jnp.float32),
            jax.ShapeDtypeStruct((B, 1), jnp.int32),
            jax.ShapeDtypeStruct((B, NCLS), jnp.float32),
        ],
        mesh=mesh,
        scratch_types=[
            pltpu.VMEM((VOCAB, PCOLS), jnp.float32),
            pltpu.VMEM((BPW, NWIDE), jnp.int32),
            pltpu.VMEM((BPW, NDEEP), jnp.int32),
            pltpu.VMEM((BPW, NCLS), jnp.float32),
            pltpu.VMEM((BPW, NCLS), jnp.float32),
            pltpu.VMEM((BPW, 1), jnp.int32),
            pltpu.VMEM((BPW, NCLS), jnp.float32),
        ],
    )
    lg, tg, pb = sc(p_tab, x_wide, x_deep, dmat)
    return (lg, tg, pb)


# speedup vs baseline: 1.2776x; 1.2776x over previous
import jax
import jax.numpy as jnp
from jax.experimental import pallas as pl
from jax.experimental.pallas import tpu as pltpu
from jax.experimental.pallas import tpu_sc as plsc

B = 16384
VOCAB = 1000
VPAD = 1024
EMB = 8
NWIDE = 8
NDEEP = 4
NDENSE = 4
NSLOT = NWIDE + NDEEP
NCLS = 2
PCOLS = NSLOT * NCLS

NC = 2
NS = 16
L = 16
NW = NC * NS
BPW = B // NW
NG = BPW // L
VROWS = VPAD // NS
VG = VROWS // L


def _rne_bf16(x):
    u = plsc.bitcast(x, jnp.int32)
    r = (u + 0x7FFF + ((u >> 16) & 1)) & jnp.int32(-65536)
    return plsc.bitcast(r, jnp.float32)


def _sc_body(emb_hbm, xw_hbm, xd_hbm, xs_hbm, wpb_hbm,
             lg_hbm, pb_hbm, tg_hbm,
             pf, xe, ps, xw, xd, xs, wpb, lg, pb, tg, pshared):
    cid = jax.lax.axis_index("c")
    sid = jax.lax.axis_index("s")
    wid = sid * NC + cid
    base = wid * BPW

    pltpu.sync_copy(emb_hbm.at[pl.ds(sid * VROWS * EMB, VROWS * EMB)], xe)
    pltpu.sync_copy(xw_hbm.at[pl.ds(base * NWIDE, BPW * NWIDE)], xw)
    pltpu.sync_copy(xd_hbm.at[pl.ds(base * NDEEP, BPW * NDEEP)], xd)
    pltpu.sync_copy(xs_hbm.at[pl.ds(base * NDENSE, BPW * NDENSE)], xs)
    pltpu.sync_copy(wpb_hbm, wpb)

    iw = jnp.arange(L, dtype=jnp.int32)

    wsp = []
    for e in range(EMB):
        r0 = _rne_bf16(wpb[pl.ds(e * PCOLS, L)])
        r1 = _rne_bf16(wpb[pl.ds(e * PCOLS + 8, L)])
        wsp.append([jnp.full((L,), r0[c], jnp.float32) for c in range(L)]
                   + [jnp.full((L,), r1[c + 8], jnp.float32) for c in range(8)])
    bsp = [jnp.full((L,), wpb[pl.ds(192, L)][c], jnp.float32) for c in range(NCLS)]
    wdr = _rne_bf16(wpb[pl.ds(208, L)])
    wdsp = [jnp.full((L,), wdr[8 + j], jnp.float32) for j in range(2 * NDENSE)]

    def vgroup(g, carry):
        vrow = g * L + iw
        eb = [_rne_bf16(plsc.load_gather(xe, [vrow * EMB + e])) for e in range(EMB)]
        for col in range(PCOLS):
            acc = eb[0] * wsp[0][col]
            for e in range(1, EMB):
                acc = acc + eb[e] * wsp[e][col]
            if col < NCLS:
                acc = acc + bsp[col]
            plsc.store_scatter(ps, [vrow * PCOLS + col], acc)
        return carry

    jax.lax.fori_loop(0, VG, vgroup, 0)
    pltpu.sync_copy(ps, pshared.at[pl.ds(sid * VROWS * PCOLS, VROWS * PCOLS)])
    plsc.subcore_barrier()
    pltpu.sync_copy(pshared, pf)

    def group(g, carry):
        row = g * L + iw
        oi = row * NCLS
        xf = [_rne_bf16(plsc.load_gather(xs, [row * NDENSE + j]))
              for j in range(NDENSE)]
        acc0 = xf[0] * wdsp[0]
        acc1 = xf[0] * wdsp[NDENSE]
        for j in range(1, NDENSE):
            acc0 = acc0 + xf[j] * wdsp[j]
            acc1 = acc1 + xf[j] * wdsp[NDENSE + j]
        for s in range(NWIDE):
            idx = plsc.load_gather(xw, [row * NWIDE + s])
            fi = idx * PCOLS + (2 * s)
            acc0 = acc0 + plsc.load_gather(pf, [fi])
            acc1 = acc1 + plsc.load_gather(pf, [fi + 1])
        for s in range(NDEEP):
            idx = plsc.load_gather(xd, [row * NDEEP + s])
            fi = idx * PCOLS + (2 * (NWIDE + s))
            acc0 = acc0 + plsc.load_gather(pf, [fi])
            acc1 = acc1 + plsc.load_gather(pf, [fi + 1])
        plsc.store_scatter(lg, [oi], acc0)
        plsc.store_scatter(lg, [oi + 1], acc1)
        m = jnp.maximum(acc0, acc1)
        e0 = jnp.exp(acc0 - m)
        e1 = jnp.exp(acc1 - m)
        inv = 1.0 / (e0 + e1)
        plsc.store_scatter(pb, [oi], e0 * inv)
        plsc.store_scatter(pb, [oi + 1], e1 * inv)
        t = jnp.where(acc1 > acc0, 1, 0).astype(jnp.int32)
        plsc.store_scatter(tg, [row], t)
        return carry

    jax.lax.fori_loop(0, NG, group, 0)

    pltpu.sync_copy(lg, lg_hbm.at[pl.ds(base * NCLS, BPW * NCLS)])
    pltpu.sync_copy(pb, pb_hbm.at[pl.ds(base * NCLS, BPW * NCLS)])
    pltpu.sync_copy(tg, tg_hbm.at[pl.ds(base, BPW)])


def kernel(x_wide, x_deep, x_dense, emb, fc_w, fc_b):
    x_wide = x_wide.astype(jnp.int32)
    x_deep = x_deep.astype(jnp.int32)
    x_dense = x_dense.astype(jnp.float32)
    emb = emb.astype(jnp.float32)
    fc_w = fc_w.astype(jnp.float32)
    fc_b = fc_b.astype(jnp.float32)

    w_proj = (
        fc_w[:, : NSLOT * EMB]
        .reshape(NCLS, NSLOT, EMB)
        .transpose(2, 1, 0)
        .reshape(EMB, PCOLS)
    )
    bias24 = jnp.concatenate([fc_b, jnp.zeros((PCOLS - NCLS,), jnp.float32)])
    wd8 = fc_w[:, NSLOT * EMB:].reshape(-1)
    wpb = jnp.concatenate([w_proj.reshape(-1), bias24, wd8])
    emb_flat = jnp.pad(emb, ((0, VPAD - VOCAB), (0, 0))).reshape(-1)

    mesh = plsc.VectorSubcoreMesh(
        core_axis_name="c", subcore_axis_name="s",
        num_cores=NC, num_subcores=NS,
    )
    sc = pl.kernel(
        _sc_body,
        compiler_params=pltpu.CompilerParams(needs_layout_passes=False),
        out_type=[
            jax.ShapeDtypeStruct((B * NCLS,), jnp.float32),
            jax.ShapeDtypeStruct((B * NCLS,), jnp.float32),
            jax.ShapeDtypeStruct((B,), jnp.int32),
        ],
        mesh=mesh,
        scratch_types=[
            pltpu.VMEM((VPAD * PCOLS,), jnp.float32),
            pltpu.VMEM((VROWS * EMB,), jnp.float32),
            pltpu.VMEM((VROWS * PCOLS,), jnp.float32),
            pltpu.VMEM((BPW * NWIDE,), jnp.int32),
            pltpu.VMEM((BPW * NDEEP,), jnp.int32),
            pltpu.VMEM((BPW * NDENSE,), jnp.float32),
            pltpu.VMEM((224,), jnp.float32),
            pltpu.VMEM((BPW * NCLS,), jnp.float32),
            pltpu.VMEM((BPW * NCLS,), jnp.float32),
            pltpu.VMEM((BPW,), jnp.int32),
            pltpu.VMEM_SHARED((VPAD * PCOLS,), jnp.float32),
        ],
    )
    lg, pb, tg = sc(
        emb_flat,
        x_wide.reshape(-1),
        x_deep.reshape(-1),
        x_dense.reshape(-1),
        wpb,
    )
    return (lg.reshape(B, NCLS), tg.reshape(B, 1), pb.reshape(B, NCLS))
